# baseline (speedup 1.0000x reference)
import functools

import jax
import jax.numpy as jnp
from jax import lax
from jax.experimental import pallas as pl
from jax.experimental.pallas import tpu as pltpu, tpu_sc as plsc

B = 16384
D = 12
HT_WORDS = 24 * 8
TAB_WORDS = 220

_info = plsc.get_sparse_core_info()
_NC, _NS, _L = _info.num_cores, _info.num_subcores, _info.num_lanes
_NW = _NC * _NS
_BPW = B // _NW
_CHUNKS = 8
_RPC = _BPW // _CHUNKS


@functools.partial(
    pl.kernel,
    mesh=plsc.VectorSubcoreMesh(core_axis_name="c", subcore_axis_name="s"),
    compiler_params=pltpu.CompilerParams(needs_layout_passes=False),
    out_type=jax.ShapeDtypeStruct((B * D,), jnp.float32),
    scratch_types=[
        pltpu.VMEM((_BPW,), jnp.int32),
        pltpu.VMEM((_BPW,), jnp.int32),
        pltpu.VMEM((TAB_WORDS,), jnp.float32),
        pltpu.VMEM((_BPW * D,), jnp.float32),
        pltpu.SemaphoreType.DMA,
        pltpu.SemaphoreType.DMA,
    ],
)
def _sc_lookup(hour_hbm, dow_hbm, tab_hbm, out_hbm,
               hour_v, dow_v, tab_v, out_v, isem, osem):
    wid = lax.axis_index("s") * _NC + lax.axis_index("c")
    base = wid * _BPW

    cp1 = pltpu.async_copy(hour_hbm.at[pl.ds(base, _BPW)], hour_v, isem)
    cp2 = pltpu.async_copy(dow_hbm.at[pl.ds(base, _BPW)], dow_v, isem)
    cp3 = pltpu.async_copy(tab_hbm, tab_v, isem)
    cp1.wait()
    cp2.wait()
    cp3.wait()

    lane = lax.iota(jnp.int32, _L)

    b_offs, cols = [], []
    for p in range(3):
        w = lane + 16 * p
        bo = w // D
        b_offs.append(bo)
        cols.append(w - bo * D)

    def chunk(k, _):
        g_lo = k * (_RPC // 4)

        @plsc.parallel_loop(g_lo, g_lo + _RPC // 4, unroll=4)
        def _(g):
            b0 = g * 4
            for p in range(3):
                bidx = b_offs[p] + b0
                h_b = plsc.load_gather(hour_v, [bidx])
                d_b = plsc.load_gather(dow_v, [bidx])
                addr = jnp.where(cols[p] < 8,
                                 h_b * 8 + cols[p],
                                 d_b * 4 + cols[p] + (HT_WORDS - 8))
                vals = plsc.load_gather(tab_v, [addr])
                out_v[pl.ds(g * 48 + p * 16, _L)] = vals

        pltpu.async_copy(
            out_v.at[pl.ds(k * _RPC * D, _RPC * D)],
            out_hbm.at[pl.ds((base + k * _RPC) * D, _RPC * D)],
            osem,
        )
        return _

    lax.fori_loop(0, _CHUNKS, chunk, None)
    for k in range(_CHUNKS):
        pltpu.make_async_copy(
            out_v.at[pl.ds(k * _RPC * D, _RPC * D)],
            out_hbm.at[pl.ds((base + k * _RPC) * D, _RPC * D)],
            osem,
        ).wait()


def kernel(hour, dow, dom, hour_table, dow_table):
    del dom
    tab = jnp.concatenate([hour_table.reshape(-1), dow_table.reshape(-1)])
    out = _sc_lookup(hour.astype(jnp.int32), dow.astype(jnp.int32), tab)
    return out.reshape(B, D)


# speedup vs baseline: 1.2313x; 1.2313x over previous
import functools

import jax
import jax.numpy as jnp
from jax import lax
from jax.experimental import pallas as pl
from jax.experimental.pallas import tpu as pltpu, tpu_sc as plsc

B = 16384
D = 12
HT_WORDS = 24 * 8
TAB_WORDS = 220

_info = plsc.get_sparse_core_info()
_NC, _NS, _L = _info.num_cores, _info.num_subcores, _info.num_lanes
_NW = _NC * _NS
_BPW = B // _NW
_CHUNKS = 8
_RPC = _BPW // _CHUNKS


@functools.partial(
    pl.kernel,
    mesh=plsc.VectorSubcoreMesh(core_axis_name="c", subcore_axis_name="s"),
    compiler_params=pltpu.CompilerParams(needs_layout_passes=False),
    out_type=jax.ShapeDtypeStruct((B, D), jnp.float32),
    scratch_types=[
        pltpu.VMEM((_BPW,), jnp.int32),
        pltpu.VMEM((_BPW,), jnp.int32),
        pltpu.VMEM((TAB_WORDS,), jnp.float32),
        pltpu.VMEM((_BPW, D), jnp.float32),
        pltpu.SemaphoreType.DMA,
        pltpu.SemaphoreType.DMA,
    ],
)
def _sc_lookup(hour_hbm, dow_hbm, ht_hbm, dt_hbm, out_hbm,
               hour_v, dow_v, tab_v, out_v, isem, osem):
    wid = lax.axis_index("s") * _NC + lax.axis_index("c")
    base = wid * _BPW

    cp1 = pltpu.async_copy(hour_hbm.at[pl.ds(base, _BPW)], hour_v, isem)
    cp2 = pltpu.async_copy(dow_hbm.at[pl.ds(base, _BPW)], dow_v, isem)
    cp3 = pltpu.async_copy(ht_hbm, tab_v.at[pl.ds(0, HT_WORDS)], isem)
    cp4 = pltpu.async_copy(dt_hbm, tab_v.at[pl.ds(HT_WORDS, 28)], isem)
    cp1.wait()
    cp2.wait()
    cp3.wait()
    cp4.wait()

    lane = lax.iota(jnp.int32, _L)

    b_offs, cols = [], []
    for p in range(3):
        w = lane + 16 * p
        bo = w // D
        b_offs.append(bo)
        cols.append(w - bo * D)

    def chunk(k, _):
        g_lo = k * (_RPC // 4)

        @plsc.parallel_loop(g_lo, g_lo + _RPC // 4, unroll=4)
        def _(g):
            b0 = g * 4
            for p in range(3):
                bidx = b_offs[p] + b0
                h_b = plsc.load_gather(hour_v, [bidx])
                d_b = plsc.load_gather(dow_v, [bidx])
                addr = jnp.where(cols[p] < 8,
                                 h_b * 8 + cols[p],
                                 d_b * 4 + cols[p] + (HT_WORDS - 8))
                vals = plsc.load_gather(tab_v, [addr])
                plsc.store_scatter(out_v, [bidx, cols[p]], vals)

        pltpu.async_copy(
            out_v.at[pl.ds(k * _RPC, _RPC)],
            out_hbm.at[pl.ds(base + k * _RPC, _RPC)],
            osem,
        )
        return _

    lax.fori_loop(0, _CHUNKS, chunk, None)
    for k in range(_CHUNKS):
        pltpu.make_async_copy(
            out_v.at[pl.ds(k * _RPC, _RPC)],
            out_hbm.at[pl.ds(base + k * _RPC, _RPC)],
            osem,
        ).wait()


def kernel(hour, dow, dom, hour_table, dow_table):
    del dom
    return _sc_lookup(hour.astype(jnp.int32), dow.astype(jnp.int32),
                      hour_table.reshape(-1), dow_table.reshape(-1))


# speedup vs baseline: 1.2640x; 1.0266x over previous
import functools

import jax
import jax.numpy as jnp
from jax import lax
from jax.experimental import pallas as pl
from jax.experimental.pallas import tpu as pltpu, tpu_sc as plsc

B = 16384
D = 12
HT_WORDS = 24 * 8
TAB_WORDS = 220

_info = plsc.get_sparse_core_info()
_NC, _NS, _L = _info.num_cores, _info.num_subcores, _info.num_lanes
_RPS = B // _NS
_R0 = 384
_R1 = _RPS - _R0
_CHUNKS = 8


@functools.partial(
    pl.kernel,
    mesh=plsc.VectorSubcoreMesh(core_axis_name="c", subcore_axis_name="s"),
    compiler_params=pltpu.CompilerParams(needs_layout_passes=False),
    out_type=jax.ShapeDtypeStruct((B, D), jnp.float32),
    scratch_types=[
        pltpu.VMEM((_R1,), jnp.int32),
        pltpu.VMEM((_R1,), jnp.int32),
        pltpu.VMEM((TAB_WORDS,), jnp.float32),
        pltpu.VMEM((_R1, D), jnp.float32),
        pltpu.SemaphoreType.DMA,
        pltpu.SemaphoreType.DMA,
    ],
)
def _sc_lookup(hour_hbm, dow_hbm, tab_hbm, out_hbm,
               hour_v, dow_v, tab_v, out_v, isem, osem):
    c = lax.axis_index("c")
    s = lax.axis_index("s")

    cpt = pltpu.async_copy(tab_hbm, tab_v, isem)

    lane = lax.iota(jnp.int32, _L)

    b_offs, cols = [], []
    for p in range(3):
        w = lane + 16 * p
        bo = w // D
        b_offs.append(bo)
        cols.append(w - bo * D)

    def run(base, rpt):
        rpc = rpt // _CHUNKS
        cp1 = pltpu.async_copy(hour_hbm.at[pl.ds(base, rpt)],
                               hour_v.at[pl.ds(0, rpt)], isem)
        cp2 = pltpu.async_copy(dow_hbm.at[pl.ds(base, rpt)],
                               dow_v.at[pl.ds(0, rpt)], isem)
        cp1.wait()
        cp2.wait()
        cpt.wait()

        def chunk(k, _):
            g_lo = k * (rpc // 4)

            @plsc.parallel_loop(g_lo, g_lo + rpc // 4, unroll=4)
            def _(g):
                b0 = g * 4
                for p in range(3):
                    bidx = b_offs[p] + b0
                    h_b = plsc.load_gather(hour_v, [bidx])
                    d_b = plsc.load_gather(dow_v, [bidx])
                    addr = jnp.where(cols[p] < 8,
                                     h_b * 8 + cols[p],
                                     d_b * 4 + cols[p] + (HT_WORDS - 8))
                    vals = plsc.load_gather(tab_v, [addr])
                    plsc.store_scatter(out_v, [bidx, cols[p]], vals)

            pltpu.async_copy(
                out_v.at[pl.ds(k * rpc, rpc)],
                out_hbm.at[pl.ds(base + k * rpc, rpc)],
                osem,
            )
            return _

        lax.fori_loop(0, _CHUNKS, chunk, None)
        for k in range(_CHUNKS):
            pltpu.make_async_copy(
                out_v.at[pl.ds(k * rpc, rpc)],
                out_hbm.at[pl.ds(base + k * rpc, rpc)],
                osem,
            ).wait()

    @pl.when(c == 0)
    def _():
        run(s * _RPS, _R0)

    @pl.when(c == 1)
    def _():
        run(s * _RPS + _R0, _R1)


def kernel(hour, dow, dom, hour_table, dow_table):
    del dom
    tab = jnp.concatenate([hour_table.reshape(-1), dow_table.reshape(-1)])
    return _sc_lookup(hour.astype(jnp.int32), dow.astype(jnp.int32), tab)


# speedup vs baseline: 1.2773x; 1.0105x over previous
import functools

import jax
import jax.numpy as jnp
from jax import lax
from jax.experimental import pallas as pl
from jax.experimental.pallas import tpu as pltpu, tpu_sc as plsc

B = 16384
D = 12
HT_WORDS = 24 * 8
TAB_WORDS = 220

_info = plsc.get_sparse_core_info()
_NC, _NS, _L = _info.num_cores, _info.num_subcores, _info.num_lanes
_RPS = B // _NS
_R0 = 320
_R1 = _RPS - _R0
_CHUNKS = 8


@functools.partial(
    pl.kernel,
    mesh=plsc.VectorSubcoreMesh(core_axis_name="c", subcore_axis_name="s"),
    compiler_params=pltpu.CompilerParams(needs_layout_passes=False),
    out_type=jax.ShapeDtypeStruct((B, D), jnp.float32),
    scratch_types=[
        pltpu.VMEM((_R1,), jnp.int32),
        pltpu.VMEM((_R1,), jnp.int32),
        pltpu.VMEM((TAB_WORDS,), jnp.float32),
        pltpu.VMEM((_R1, D), jnp.float32),
        pltpu.SemaphoreType.DMA,
        pltpu.SemaphoreType.DMA,
    ],
)
def _sc_lookup(hour_hbm, dow_hbm, tab_hbm, out_hbm,
               hour_v, dow_v, tab_v, out_v, isem, osem):
    c = lax.axis_index("c")
    s = lax.axis_index("s")

    cpt = pltpu.async_copy(tab_hbm, tab_v, isem)

    lane = lax.iota(jnp.int32, _L)

    b_offs, cols = [], []
    for p in range(3):
        w = lane + 16 * p
        bo = w // D
        b_offs.append(bo)
        cols.append(w - bo * D)

    def run(base, rpt):
        rpc = rpt // _CHUNKS
        cp1 = pltpu.async_copy(hour_hbm.at[pl.ds(base, rpt)],
                               hour_v.at[pl.ds(0, rpt)], isem)
        cp2 = pltpu.async_copy(dow_hbm.at[pl.ds(base, rpt)],
                               dow_v.at[pl.ds(0, rpt)], isem)
        cp1.wait()
        cp2.wait()
        cpt.wait()

        def chunk(k, _):
            g_lo = k * (rpc // 4)

            @plsc.parallel_loop(g_lo, g_lo + rpc // 4, unroll=4)
            def _(g):
                b0 = g * 4
                for p in range(3):
                    bidx = b_offs[p] + b0
                    h_b = plsc.load_gather(hour_v, [bidx])
                    d_b = plsc.load_gather(dow_v, [bidx])
                    addr = jnp.where(cols[p] < 8,
                                     h_b * 8 + cols[p],
                                     d_b * 4 + cols[p] + (HT_WORDS - 8))
                    vals = plsc.load_gather(tab_v, [addr])
                    plsc.store_scatter(out_v, [bidx, cols[p]], vals)

            pltpu.async_copy(
                out_v.at[pl.ds(k * rpc, rpc)],
                out_hbm.at[pl.ds(base + k * rpc, rpc)],
                osem,
            )
            return _

        lax.fori_loop(0, _CHUNKS, chunk, None)
        for k in range(_CHUNKS):
            pltpu.make_async_copy(
                out_v.at[pl.ds(k * rpc, rpc)],
                out_hbm.at[pl.ds(base + k * rpc, rpc)],
                osem,
            ).wait()

    @pl.when(c == 0)
    def _():
        run(s * _RPS, _R0)

    @pl.when(c == 1)
    def _():
        run(s * _RPS + _R0, _R1)


def kernel(hour, dow, dom, hour_table, dow_table):
    del dom
    tab = jnp.concatenate([hour_table.reshape(-1), dow_table.reshape(-1)])
    return _sc_lookup(hour.astype(jnp.int32), dow.astype(jnp.int32), tab)


# speedup vs baseline: 1.2890x; 1.0092x over previous
import functools

import jax
import jax.numpy as jnp
from jax import lax
from jax.experimental import pallas as pl
from jax.experimental.pallas import tpu as pltpu, tpu_sc as plsc

B = 16384
D = 12
HT_WORDS = 24 * 8
TAB_WORDS = 220

_info = plsc.get_sparse_core_info()
_NC, _NS, _L = _info.num_cores, _info.num_subcores, _info.num_lanes
_RPS = B // _NS
_R0 = 320
_R1 = _RPS - _R0
_CHUNKS = 8


@functools.partial(
    pl.kernel,
    mesh=plsc.VectorSubcoreMesh(core_axis_name="c", subcore_axis_name="s"),
    compiler_params=pltpu.CompilerParams(needs_layout_passes=False),
    out_type=jax.ShapeDtypeStruct((B, D), jnp.float32),
    scratch_types=[
        pltpu.VMEM((_R1,), jnp.int32),
        pltpu.VMEM((_R1,), jnp.int32),
        pltpu.VMEM((TAB_WORDS,), jnp.float32),
        pltpu.VMEM((_R1, D), jnp.float32),
        pltpu.SemaphoreType.DMA,
        pltpu.SemaphoreType.DMA,
    ],
)
def _sc_lookup(hour_hbm, dow_hbm, tab_hbm, out_hbm,
               hour_v, dow_v, tab_v, out_v, isem, osem):
    c = lax.axis_index("c")
    s = lax.axis_index("s")

    cpt = pltpu.async_copy(tab_hbm, tab_v, isem)

    lane = lax.iota(jnp.int32, _L)

    b_offs, cols = [], []
    for p in range(3):
        w = lane + 16 * p
        bo = w // D
        b_offs.append(bo)
        cols.append(w - bo * D)

    def run(base, rpt):
        rpc = rpt // _CHUNKS
        cp1 = pltpu.async_copy(hour_hbm.at[pl.ds(base, rpt)],
                               hour_v.at[pl.ds(0, rpt)], isem)
        cp2 = pltpu.async_copy(dow_hbm.at[pl.ds(base, rpt)],
                               dow_v.at[pl.ds(0, rpt)], isem)
        cp1.wait()
        cp2.wait()
        cpt.wait()

        def chunk(k, _):
            g_lo = k * (rpc // 4)

            @plsc.parallel_loop(g_lo, g_lo + rpc // 4, unroll=8)
            def _(g):
                b0 = g * 4
                for p in range(3):
                    bidx = b_offs[p] + b0
                    h_b = plsc.load_gather(hour_v, [bidx])
                    d_b = plsc.load_gather(dow_v, [bidx])
                    addr = jnp.where(cols[p] < 8,
                                     h_b * 8 + cols[p],
                                     d_b * 4 + cols[p] + (HT_WORDS - 8))
                    vals = plsc.load_gather(tab_v, [addr])
                    plsc.store_scatter(out_v, [bidx, cols[p]], vals)

            pltpu.async_copy(
                out_v.at[pl.ds(k * rpc, rpc)],
                out_hbm.at[pl.ds(base + k * rpc, rpc)],
                osem,
            )
            return _

        lax.fori_loop(0, _CHUNKS, chunk, None)
        for k in range(_CHUNKS):
            pltpu.make_async_copy(
                out_v.at[pl.ds(k * rpc, rpc)],
                out_hbm.at[pl.ds(base + k * rpc, rpc)],
                osem,
            ).wait()

    @pl.when(c == 0)
    def _():
        run(s * _RPS, _R0)

    @pl.when(c == 1)
    def _():
        run(s * _RPS + _R0, _R1)


def kernel(hour, dow, dom, hour_table, dow_table):
    del dom
    tab = jnp.concatenate([hour_table.reshape(-1), dow_table.reshape(-1)])
    return _sc_lookup(hour.astype(jnp.int32), dow.astype(jnp.int32), tab)
